# Initial kernel scaffold; baseline (speedup 1.0000x reference)
import jax
import jax.numpy as jnp
from jax.experimental import pallas as pl


def kernel(x, xyz, knn, g_pos, ne_W1, ne_g1, ne_b1, ne_W2, ne_g2, ne_b2, ne_W3, nbr_g, nbr_b, gpe_W, mlp_Wi, mlp_bi, mlp_Wo, mlp_g, mlp_b, lfa_proj, lfa_g, lfa_b, nca_W1, nca_b1, nca_W2, nca_b2, nca_W3a, nca_b3a, nca_W3b, nca_b3b, mlps_Wi, mlps_bi, mlps_Wo, mlps_g, mlps_b, post_g, post_b, post_W):
    raise NotImplementedError("write your pallas kernel here")


# speedup vs baseline: 1.6167x; 1.6167x over previous
import functools
import math

import jax
import jax.numpy as jnp
from jax import lax
from jax.experimental import pallas as pl
from jax.experimental.pallas import tpu as pltpu
from jax.experimental.pallas import tpu_sc as plsc

N = 10000
K = 16
D = 128
NP = 10240
E = NP * K
B = 256
BK = B * K
GRID = NP // B

_BN_S = 1.0 / math.sqrt(1.0 + 1e-5)
_INV_SQRT2 = 1.0 / math.sqrt(2.0)


def _gelu(x):
    return 0.5 * x * (1.0 + lax.erf(x * _INV_SQRT2))


def _bn(x, g_ref, b_ref):
    return x * (g_ref[...] * _BN_S) + b_ref[...]


def _sc_gather(table, idx, C):
    info = plsc.get_sparse_core_info()
    nw = info.num_cores * info.num_subcores
    b_per_w = E // nw
    ch = 128
    n_ch = b_per_w // ch
    mesh = plsc.VectorSubcoreMesh(core_axis_name="c", subcore_axis_name="s")

    def body(table_hbm, idx_hbm, out_hbm, idx_v, rows_v, sem):
        wid = lax.axis_index("s") * info.num_cores + lax.axis_index("c")
        base = wid * b_per_w

        def step(c, carry):
            off = base + c * ch
            pltpu.sync_copy(idx_hbm.at[pl.ds(off, ch)], idx_v)
            pltpu.async_copy(table_hbm.at[idx_v], rows_v, sem).wait()
            pltpu.sync_copy(rows_v, out_hbm.at[pl.ds(off, ch)])
            return carry

        lax.fori_loop(0, n_ch, step, 0)

    fn = pl.kernel(
        body,
        out_type=jax.ShapeDtypeStruct((E, C), jnp.float32),
        mesh=mesh,
        scratch_types=[
            pltpu.VMEM((ch,), jnp.int32),
            pltpu.VMEM((ch, C), jnp.float32),
            pltpu.SemaphoreType.DMA,
        ],
    )
    return fn(table, idx)


def _rep(x, c):
    return jnp.broadcast_to(x[:, None, :], (B, K, c)).reshape(BK, c)


def _pre_body(g10, tbl, gpos, W1s, W1d, g1, b1, W2, g2, b2, W3, nbrg, nbrb,
              gpeW, mWi, mbi, mWo, mg, mb, proj0, h_out, xp_out):
    e = g10[...]
    h1 = jnp.dot(e, W1s[...], preferred_element_type=jnp.float32)
    dd = jnp.dot(tbl[...], W1d[...], preferred_element_type=jnp.float32)
    h1 = h1 - _rep(dd, 16)
    h1 = _gelu(_bn(h1, g1, b1))
    h2 = jnp.dot(h1, W2[...], preferred_element_type=jnp.float32)
    h2 = _gelu(_bn(h2, g2, b2))
    h3 = jnp.dot(h2, W3[...], preferred_element_type=jnp.float32)
    s3 = h3.reshape(B, K, D)
    feat = s3[:, 0, :]
    for j in range(1, K):
        feat = jnp.maximum(feat, s3[:, j, :])
    feat = _bn(feat, nbrg, nbrb)
    h = feat + jnp.dot(gpos[...], gpeW[...], preferred_element_type=jnp.float32)
    t = _gelu(jnp.dot(h, mWi[...], preferred_element_type=jnp.float32) + mbi[...])
    h = h + _bn(jnp.dot(t, mWo[...], preferred_element_type=jnp.float32), mg, mb)
    h_out[...] = h
    xp_out[...] = jnp.dot(h, proj0[...], preferred_element_type=jnp.float32)


def _full(a):
    return pl.BlockSpec(a.shape, lambda i: (0,) * a.ndim)


def _pre_call(g10, tbl, gpos, *ws):
    specs = [
        pl.BlockSpec((BK, 16), lambda i: (i, 0)),
        pl.BlockSpec((B, 16), lambda i: (i, 0)),
        pl.BlockSpec((B, 64), lambda i: (i, 0)),
    ] + [_full(w) for w in ws]
    out_spec = pl.BlockSpec((B, D), lambda i: (i, 0))
    return pl.pallas_call(
        _pre_body,
        grid=(GRID,),
        in_specs=specs,
        out_specs=[out_spec, out_spec],
        out_shape=[jax.ShapeDtypeStruct((NP, D), jnp.float32)] * 2,
    )(g10, tbl, gpos, *ws)


def _lfa_body(has_mlp, is_last, h_in, xp_in, gxp, g10, tbl,
              W1s, W1d, b1, W2i, b2, W3at, W3ab, b3a, W3b, b3b, lg, lb,
              *rest):
    if has_mlp:
        msWi, msbi, msWo, msg, msb = rest[:5]
        rest = rest[5:]
    if is_last:
        postg, postb, postW = rest[:3]
        out_ref = rest[3]
    else:
        projn = rest[0]
        h_out, xp_out = rest[1], rest[2]

    e = g10[...]
    p0 = jnp.dot(e, W1s[...], preferred_element_type=jnp.float32) + b1[...]
    dd = jnp.dot(tbl[...], W1d[...], preferred_element_type=jnp.float32)
    p0 = p0 - _rep(dd, 64)
    p03 = p0.reshape(B, K, 64)
    plo = p03[:, 0, :]
    for j in range(1, K):
        plo = jnp.maximum(plo, p03[:, j, :])
    p1 = jnp.dot(p0, W2i[...], preferred_element_type=jnp.float32) + b2[...]
    q = jnp.dot(plo, W3ab[...], preferred_element_type=jnp.float32)
    a = jnp.dot(p1, W3at[...], preferred_element_type=jnp.float32)
    a = a + _rep(q, D) + b3a[...]
    pe = jnp.dot(_gelu(a), W3b[...], preferred_element_type=jnp.float32) + b3b[...]
    s = (gxp[...] + pe).reshape(B, K, D)
    m = s[:, 0, :]
    for j in range(1, K):
        m = jnp.maximum(m, s[:, j, :])
    h = h_in[...] + _bn(m - xp_in[...], lg, lb)
    if has_mlp:
        t = _gelu(jnp.dot(h, msWi[...], preferred_element_type=jnp.float32) + msbi[...])
        h = h + _bn(jnp.dot(t, msWo[...], preferred_element_type=jnp.float32), msg, msb)
    if is_last:
        out_ref[...] = jnp.dot(_bn(h, postg, postb), postW[...],
                               preferred_element_type=jnp.float32)
    else:
        h_out[...] = h
        xp_out[...] = jnp.dot(h, projn[...], preferred_element_type=jnp.float32)


def _lfa_call(has_mlp, is_last, h, xp, gxp, g10, tbl, *ws):
    specs = [
        pl.BlockSpec((B, D), lambda i: (i, 0)),
        pl.BlockSpec((B, D), lambda i: (i, 0)),
        pl.BlockSpec((BK, D), lambda i: (i, 0)),
        pl.BlockSpec((BK, 16), lambda i: (i, 0)),
        pl.BlockSpec((B, 16), lambda i: (i, 0)),
    ] + [_full(w) for w in ws]
    out_spec = pl.BlockSpec((B, D), lambda i: (i, 0))
    if is_last:
        out_specs = out_spec
        out_shape = jax.ShapeDtypeStruct((NP, D), jnp.float32)
    else:
        out_specs = [out_spec, out_spec]
        out_shape = [jax.ShapeDtypeStruct((NP, D), jnp.float32)] * 2
    return pl.pallas_call(
        functools.partial(_lfa_body, has_mlp, is_last),
        grid=(GRID,),
        in_specs=specs,
        out_specs=out_specs,
        out_shape=out_shape,
    )(h, xp, gxp, g10, tbl, *ws)


def kernel(x, xyz, knn, g_pos, ne_W1, ne_g1, ne_b1, ne_W2, ne_g2, ne_b2,
           ne_W3, nbr_g, nbr_b, gpe_W, mlp_Wi, mlp_bi, mlp_Wo, mlp_g, mlp_b,
           lfa_proj, lfa_g, lfa_b, nca_W1, nca_b1, nca_W2, nca_b2, nca_W3a,
           nca_b3a, nca_W3b, nca_b3b, mlps_Wi, mlps_bi, mlps_Wo, mlps_g,
           mlps_b, post_g, post_b, post_W):
    f32 = jnp.float32
    padn = NP - N
    tbl = jnp.concatenate([xyz, x, jnp.zeros((N, 6), f32)], axis=1)
    tbl = jnp.pad(tbl, ((0, padn), (0, 0)))
    knnf = jnp.pad(knn, ((0, padn), (0, 0))).reshape(E).astype(jnp.int32)
    gpos = jnp.pad(g_pos, ((0, padn), (0, 0)))

    r = lambda v: v.reshape(1, -1)

    tblw = jnp.pad(tbl, ((0, 0), (0, D - 16)))
    g10 = _sc_gather(tblw, knnf, D)[:, :16]

    W1s = jnp.pad(ne_W1, ((0, 6), (0, 0)))
    W1d = jnp.pad(ne_W1[:3], ((0, 13), (0, 0)))
    h, xp = _pre_call(
        g10, tbl, gpos, W1s, W1d, r(ne_g1), r(ne_b1), ne_W2, r(ne_g2),
        r(ne_b2), ne_W3, r(nbr_g), r(nbr_b), gpe_W, mlp_Wi, r(mlp_bi),
        mlp_Wo, r(mlp_g), r(mlp_b), lfa_proj[0])

    out = None
    for i in range(4):
        gxp = _sc_gather(xp, knnf, D)
        nW1s = jnp.pad(nca_W1[i], ((0, 13), (0, 0)))
        args = [h, xp, gxp, g10, tbl, nW1s, nW1s, r(nca_b1[i]), nca_W2[i],
                r(nca_b2[i]), nca_W3a[i][:64], nca_W3a[i][64:],
                r(nca_b3a[i]), nca_W3b[i], r(nca_b3b[i]), r(lfa_g[i]),
                r(lfa_b[i])]
        has_mlp = i % 2 == 1
        is_last = i == 3
        if has_mlp:
            j = i // 2
            args += [mlps_Wi[j], r(mlps_bi[j]), mlps_Wo[j], r(mlps_g[j]),
                     r(mlps_b[j])]
        if is_last:
            args += [r(post_g), r(post_b), post_W]
            out = _lfa_call(has_mlp, is_last, *args)
        else:
            args += [lfa_proj[i + 1]]
            h, xp = _lfa_call(has_mlp, is_last, *args)

    return out[:N]


# speedup vs baseline: 1.9161x; 1.1852x over previous
import functools
import math

import jax
import jax.numpy as jnp
from jax import lax
from jax.experimental import pallas as pl
from jax.experimental.pallas import tpu as pltpu
from jax.experimental.pallas import tpu_sc as plsc

N = 10000
K = 16
D = 128
NP = 10240
E = NP * K
B = 256
BK = B * K
GRID = NP // B

_BN_S = 1.0 / math.sqrt(1.0 + 1e-5)
_INV_SQRT2 = 1.0 / math.sqrt(2.0)


def _gelu(x):
    return 0.5 * x * (1.0 + lax.erf(x * _INV_SQRT2))


def _bn(x, g_ref, b_ref):
    return x * (g_ref[...] * _BN_S) + b_ref[...]


def _sc_gather(table, idx2d, C):
    info = plsc.get_sparse_core_info()
    nw = info.num_cores * info.num_subcores
    b_per_w = E // nw
    ch = 128
    n_ch = b_per_w // ch
    nb = 4
    n_grp = n_ch // nb
    mesh = plsc.VectorSubcoreMesh(core_axis_name="c", subcore_axis_name="s")

    def body(table_hbm, idx_hbm, out_hbm, idx_v, rows_v, sem_g, sem_o):
        wid = lax.axis_index("s") * info.num_cores + lax.axis_index("c")
        base = wid * b_per_w
        pltpu.sync_copy(idx_hbm.at[pl.ds(wid * n_ch, n_ch)], idx_v)
        for b in range(nb):
            pltpu.async_copy(table_hbm.at[idx_v.at[b]], rows_v.at[b],
                             sem_g.at[b])

        def grp(g, carry):
            for b in range(nb):
                c = g * nb + b
                pltpu.make_async_copy(table_hbm.at[idx_v.at[b]],
                                      rows_v.at[b], sem_g.at[b]).wait()
                pltpu.async_copy(rows_v.at[b],
                                 out_hbm.at[pl.ds(base + c * ch, ch)],
                                 sem_o.at[b])

                @pl.when(g < n_grp - 1)
                def _():
                    pltpu.make_async_copy(
                        rows_v.at[b], out_hbm.at[pl.ds(base, ch)],
                        sem_o.at[b]).wait()
                    pltpu.async_copy(table_hbm.at[idx_v.at[c + nb]],
                                     rows_v.at[b], sem_g.at[b])
            return carry

        lax.fori_loop(0, n_grp, grp, 0)
        for b in range(nb):
            pltpu.make_async_copy(rows_v.at[b], out_hbm.at[pl.ds(base, ch)],
                                  sem_o.at[b]).wait()

    fn = pl.kernel(
        body,
        out_type=jax.ShapeDtypeStruct((E, C), jnp.float32),
        mesh=mesh,
        scratch_types=[
            pltpu.VMEM((n_ch, ch), jnp.int32),
            pltpu.VMEM((nb, ch, C), jnp.float32),
            pltpu.SemaphoreType.DMA((nb,)),
            pltpu.SemaphoreType.DMA((nb,)),
        ],
    )
    return fn(table, idx2d)


def _rep(x, c):
    return jnp.broadcast_to(x[:, None, :], (B, K, c)).reshape(BK, c)


def _pre_body(g10, tbl, gpos, W1s, W1d, g1, b1, W2, g2, b2, W3, nbrg, nbrb,
              gpeW, mWi, mbi, mWo, mg, mb, proj0, h_out, xp_out):
    e = g10[...]
    h1 = jnp.dot(e, W1s[...], preferred_element_type=jnp.float32)
    dd = jnp.dot(tbl[...], W1d[...], preferred_element_type=jnp.float32)
    h1 = h1 - _rep(dd, 16)
    h1 = _gelu(_bn(h1, g1, b1))
    h2 = jnp.dot(h1, W2[...], preferred_element_type=jnp.float32)
    h2 = _gelu(_bn(h2, g2, b2))
    h3 = jnp.dot(h2, W3[...], preferred_element_type=jnp.float32)
    s3 = h3.reshape(B, K, D)
    feat = s3[:, 0, :]
    for j in range(1, K):
        feat = jnp.maximum(feat, s3[:, j, :])
    feat = _bn(feat, nbrg, nbrb)
    h = feat + jnp.dot(gpos[...], gpeW[...], preferred_element_type=jnp.float32)
    t = _gelu(jnp.dot(h, mWi[...], preferred_element_type=jnp.float32) + mbi[...])
    h = h + _bn(jnp.dot(t, mWo[...], preferred_element_type=jnp.float32), mg, mb)
    h_out[...] = h
    xp_out[...] = jnp.dot(h, proj0[...], preferred_element_type=jnp.float32)


def _full(a):
    return pl.BlockSpec(a.shape, lambda i: (0,) * a.ndim)


def _pre_call(g10, tbl, gpos, *ws):
    specs = [
        pl.BlockSpec((BK, 16), lambda i: (i, 0)),
        pl.BlockSpec((B, 16), lambda i: (i, 0)),
        pl.BlockSpec((B, 64), lambda i: (i, 0)),
    ] + [_full(w) for w in ws]
    out_spec = pl.BlockSpec((B, D), lambda i: (i, 0))
    return pl.pallas_call(
        _pre_body,
        grid=(GRID,),
        in_specs=specs,
        out_specs=[out_spec, out_spec],
        out_shape=[jax.ShapeDtypeStruct((NP, D), jnp.float32)] * 2,
    )(g10, tbl, gpos, *ws)


def _lfa_body(has_mlp, is_last, h_in, xp_in, gxp, g10, tbl,
              W1s, W1d, b1, W2i, b2, W3at, W3ab, b3a, W3b, b3b, lg, lb,
              *rest):
    if has_mlp:
        msWi, msbi, msWo, msg, msb = rest[:5]
        rest = rest[5:]
    if is_last:
        postg, postb, postW = rest[:3]
        out_ref = rest[3]
    else:
        projn = rest[0]
        h_out, xp_out = rest[1], rest[2]

    e = g10[...]
    p0 = jnp.dot(e, W1s[...], preferred_element_type=jnp.float32) + b1[...]
    dd = jnp.dot(tbl[...], W1d[...], preferred_element_type=jnp.float32)
    p0 = p0 - _rep(dd, 64)
    p03 = p0.reshape(B, K, 64)
    plo = p03[:, 0, :]
    for j in range(1, K):
        plo = jnp.maximum(plo, p03[:, j, :])
    p1 = jnp.dot(p0, W2i[...], preferred_element_type=jnp.float32) + b2[...]
    q = jnp.dot(plo, W3ab[...], preferred_element_type=jnp.float32)
    a = jnp.dot(p1, W3at[...], preferred_element_type=jnp.float32)
    a = a + _rep(q, D) + b3a[...]
    pe = jnp.dot(_gelu(a), W3b[...], preferred_element_type=jnp.float32) + b3b[...]
    s = (gxp[...] + pe).reshape(B, K, D)
    m = s[:, 0, :]
    for j in range(1, K):
        m = jnp.maximum(m, s[:, j, :])
    h = h_in[...] + _bn(m - xp_in[...], lg, lb)
    if has_mlp:
        t = _gelu(jnp.dot(h, msWi[...], preferred_element_type=jnp.float32) + msbi[...])
        h = h + _bn(jnp.dot(t, msWo[...], preferred_element_type=jnp.float32), msg, msb)
    if is_last:
        out_ref[...] = jnp.dot(_bn(h, postg, postb), postW[...],
                               preferred_element_type=jnp.float32)
    else:
        h_out[...] = h
        xp_out[...] = jnp.dot(h, projn[...], preferred_element_type=jnp.float32)


def _lfa_call(has_mlp, is_last, h, xp, gxp, g10, tbl, *ws):
    specs = [
        pl.BlockSpec((B, D), lambda i: (i, 0)),
        pl.BlockSpec((B, D), lambda i: (i, 0)),
        pl.BlockSpec((BK, D), lambda i: (i, 0)),
        pl.BlockSpec((BK, 16), lambda i: (i, 0)),
        pl.BlockSpec((B, 16), lambda i: (i, 0)),
    ] + [_full(w) for w in ws]
    out_spec = pl.BlockSpec((B, D), lambda i: (i, 0))
    if is_last:
        out_specs = out_spec
        out_shape = jax.ShapeDtypeStruct((NP, D), jnp.float32)
    else:
        out_specs = [out_spec, out_spec]
        out_shape = [jax.ShapeDtypeStruct((NP, D), jnp.float32)] * 2
    return pl.pallas_call(
        functools.partial(_lfa_body, has_mlp, is_last),
        grid=(GRID,),
        in_specs=specs,
        out_specs=out_specs,
        out_shape=out_shape,
    )(h, xp, gxp, g10, tbl, *ws)


def kernel(x, xyz, knn, g_pos, ne_W1, ne_g1, ne_b1, ne_W2, ne_g2, ne_b2,
           ne_W3, nbr_g, nbr_b, gpe_W, mlp_Wi, mlp_bi, mlp_Wo, mlp_g, mlp_b,
           lfa_proj, lfa_g, lfa_b, nca_W1, nca_b1, nca_W2, nca_b2, nca_W3a,
           nca_b3a, nca_W3b, nca_b3b, mlps_Wi, mlps_bi, mlps_Wo, mlps_g,
           mlps_b, post_g, post_b, post_W):
    f32 = jnp.float32
    padn = NP - N
    tbl = jnp.concatenate([xyz, x, jnp.zeros((N, 6), f32)], axis=1)
    tbl = jnp.pad(tbl, ((0, padn), (0, 0)))
    knnf = jnp.pad(knn, ((0, padn), (0, 0))).reshape(E // 128, 128).astype(jnp.int32)
    gpos = jnp.pad(g_pos, ((0, padn), (0, 0)))

    r = lambda v: v.reshape(1, -1)

    tblw = jnp.pad(tbl, ((0, 0), (0, D - 16)))
    g10 = _sc_gather(tblw, knnf, D)[:, :16]

    W1s = jnp.pad(ne_W1, ((0, 6), (0, 0)))
    W1d = jnp.pad(ne_W1[:3], ((0, 13), (0, 0)))
    h, xp = _pre_call(
        g10, tbl, gpos, W1s, W1d, r(ne_g1), r(ne_b1), ne_W2, r(ne_g2),
        r(ne_b2), ne_W3, r(nbr_g), r(nbr_b), gpe_W, mlp_Wi, r(mlp_bi),
        mlp_Wo, r(mlp_g), r(mlp_b), lfa_proj[0])

    out = None
    for i in range(4):
        gxp = _sc_gather(xp, knnf, D)
        nW1s = jnp.pad(nca_W1[i], ((0, 13), (0, 0)))
        args = [h, xp, gxp, g10, tbl, nW1s, nW1s, r(nca_b1[i]), nca_W2[i],
                r(nca_b2[i]), nca_W3a[i][:64], nca_W3a[i][64:],
                r(nca_b3a[i]), nca_W3b[i], r(nca_b3b[i]), r(lfa_g[i]),
                r(lfa_b[i])]
        has_mlp = i % 2 == 1
        is_last = i == 3
        if has_mlp:
            j = i // 2
            args += [mlps_Wi[j], r(mlps_bi[j]), mlps_Wo[j], r(mlps_g[j]),
                     r(mlps_b[j])]
        if is_last:
            args += [r(post_g), r(post_b), post_W]
            out = _lfa_call(has_mlp, is_last, *args)
        else:
            args += [lfa_proj[i + 1]]
            h, xp = _lfa_call(has_mlp, is_last, *args)

    return out[:N]


# speedup vs baseline: 2.2318x; 1.1648x over previous
import functools
import math

import jax
import jax.numpy as jnp
from jax import lax
from jax.experimental import pallas as pl
from jax.experimental.pallas import tpu as pltpu
from jax.experimental.pallas import tpu_sc as plsc

N = 10000
K = 16
D = 128
NP = 10240
E = NP * K
B = 256
BK = B * K
GRID = NP // B

_BN_S = 1.0 / math.sqrt(1.0 + 1e-5)
_INV_SQRT2 = 1.0 / math.sqrt(2.0)


def _gelu(x):
    return 0.5 * x * (1.0 + lax.erf(x * _INV_SQRT2))


def _bn(x, g_ref, b_ref):
    return x * (g_ref[...] * _BN_S) + b_ref[...]


def _sc_gather(table, idx2d, C):
    info = plsc.get_sparse_core_info()
    nw = info.num_cores * info.num_subcores
    b_per_w = E // nw
    ch = 80
    n_ch = b_per_w // ch
    nb = 8
    n_grp = n_ch // nb
    mesh = plsc.VectorSubcoreMesh(core_axis_name="c", subcore_axis_name="s")

    def body(table_hbm, idx_hbm, out_hbm, idx_v, rows_v, sem_g, sem_o):
        wid = lax.axis_index("s") * info.num_cores + lax.axis_index("c")
        base = wid * b_per_w
        pltpu.sync_copy(idx_hbm.at[pl.ds(wid * n_ch, n_ch)], idx_v)
        for b in range(nb):
            pltpu.async_copy(table_hbm.at[idx_v.at[b]], rows_v.at[b],
                             sem_g.at[b])

        def grp(g, carry):
            for b in range(nb):
                c = g * nb + b
                pltpu.make_async_copy(table_hbm.at[idx_v.at[b]],
                                      rows_v.at[b], sem_g.at[b]).wait()
                pltpu.async_copy(rows_v.at[b],
                                 out_hbm.at[pl.ds(base + c * ch, ch)],
                                 sem_o.at[b])

                @pl.when(g < n_grp - 1)
                def _():
                    pltpu.make_async_copy(
                        rows_v.at[b], out_hbm.at[pl.ds(base, ch)],
                        sem_o.at[b]).wait()
                    pltpu.async_copy(table_hbm.at[idx_v.at[c + nb]],
                                     rows_v.at[b], sem_g.at[b])
            return carry

        lax.fori_loop(0, n_grp, grp, 0)
        for b in range(nb):
            pltpu.make_async_copy(rows_v.at[b], out_hbm.at[pl.ds(base, ch)],
                                  sem_o.at[b]).wait()

    fn = pl.kernel(
        body,
        out_type=jax.ShapeDtypeStruct((E, C), jnp.float32),
        mesh=mesh,
        scratch_types=[
            pltpu.VMEM((n_ch, ch), jnp.int32),
            pltpu.VMEM((nb, ch, C), jnp.float32),
            pltpu.SemaphoreType.DMA((nb,)),
            pltpu.SemaphoreType.DMA((nb,)),
        ],
    )
    return fn(table, idx2d)


def _rep(x, c):
    return jnp.broadcast_to(x[:, None, :], (B, K, c)).reshape(BK, c)


def _pre_body(g10, tbl, gpos, W1s, W1d, g1, b1, W2, g2, b2, W3, nbrg, nbrb,
              gpeW, mWi, mbi, mWo, mg, mb, proj0, h_out, xp_out):
    e = g10[...]
    h1 = jnp.dot(e, W1s[...], preferred_element_type=jnp.float32)
    dd = jnp.dot(tbl[...], W1d[...], preferred_element_type=jnp.float32)
    h1 = h1 - _rep(dd, 16)
    h1 = _gelu(_bn(h1, g1, b1))
    h2 = jnp.dot(h1, W2[...], preferred_element_type=jnp.float32)
    h2 = _gelu(_bn(h2, g2, b2))
    h3 = jnp.dot(h2, W3[...], preferred_element_type=jnp.float32)
    feat = jnp.max(h3.reshape(B, K, D), axis=1)
    feat = _bn(feat, nbrg, nbrb)
    h = feat + jnp.dot(gpos[...], gpeW[...], preferred_element_type=jnp.float32)
    t = _gelu(jnp.dot(h, mWi[...], preferred_element_type=jnp.float32) + mbi[...])
    h = h + _bn(jnp.dot(t, mWo[...], preferred_element_type=jnp.float32), mg, mb)
    h_out[...] = h
    xp_out[...] = jnp.dot(h, proj0[...], preferred_element_type=jnp.float32)


def _full(a):
    return pl.BlockSpec(a.shape, lambda i: (0,) * a.ndim)


def _pre_call(g10, tbl, gpos, *ws):
    specs = [
        pl.BlockSpec((BK, 16), lambda i: (i, 0)),
        pl.BlockSpec((B, 16), lambda i: (i, 0)),
        pl.BlockSpec((B, 64), lambda i: (i, 0)),
    ] + [_full(w) for w in ws]
    out_spec = pl.BlockSpec((B, D), lambda i: (i, 0))
    return pl.pallas_call(
        _pre_body,
        grid=(GRID,),
        in_specs=specs,
        out_specs=[out_spec, out_spec],
        out_shape=[jax.ShapeDtypeStruct((NP, D), jnp.float32)] * 2,
    )(g10, tbl, gpos, *ws)


def _lfa_body(has_mlp, is_last, h_in, xp_in, gxp, g10, tbl,
              W1s, W1d, b1, W2i, b2, W3at, W3ab, b3a, W3b, b3b, lg, lb,
              *rest):
    if has_mlp:
        msWi, msbi, msWo, msg, msb = rest[:5]
        rest = rest[5:]
    if is_last:
        postg, postb, postW = rest[:3]
        out_ref = rest[3]
    else:
        projn = rest[0]
        h_out, xp_out = rest[1], rest[2]

    e = g10[...]
    p0 = jnp.dot(e, W1s[...], preferred_element_type=jnp.float32) + b1[...]
    dd = jnp.dot(tbl[...], W1d[...], preferred_element_type=jnp.float32)
    p0 = p0 - _rep(dd, 64)
    plo = jnp.max(p0.reshape(B, K, 64), axis=1)
    p1 = jnp.dot(p0, W2i[...], preferred_element_type=jnp.float32) + b2[...]
    q = jnp.dot(plo, W3ab[...], preferred_element_type=jnp.float32)
    a = jnp.dot(p1, W3at[...], preferred_element_type=jnp.float32)
    a = a + _rep(q, D) + b3a[...]
    pe = jnp.dot(_gelu(a), W3b[...], preferred_element_type=jnp.float32) + b3b[...]
    m = jnp.max((gxp[...] + pe).reshape(B, K, D), axis=1)
    h = h_in[...] + _bn(m - xp_in[...], lg, lb)
    if has_mlp:
        t = _gelu(jnp.dot(h, msWi[...], preferred_element_type=jnp.float32) + msbi[...])
        h = h + _bn(jnp.dot(t, msWo[...], preferred_element_type=jnp.float32), msg, msb)
    if is_last:
        out_ref[...] = jnp.dot(_bn(h, postg, postb), postW[...],
                               preferred_element_type=jnp.float32)
    else:
        h_out[...] = h
        xp_out[...] = jnp.dot(h, projn[...], preferred_element_type=jnp.float32)


def _lfa_call(has_mlp, is_last, h, xp, gxp, g10, tbl, *ws):
    specs = [
        pl.BlockSpec((B, D), lambda i: (i, 0)),
        pl.BlockSpec((B, D), lambda i: (i, 0)),
        pl.BlockSpec((BK, D), lambda i: (i, 0)),
        pl.BlockSpec((BK, 16), lambda i: (i, 0)),
        pl.BlockSpec((B, 16), lambda i: (i, 0)),
    ] + [_full(w) for w in ws]
    out_spec = pl.BlockSpec((B, D), lambda i: (i, 0))
    if is_last:
        out_specs = out_spec
        out_shape = jax.ShapeDtypeStruct((NP, D), jnp.float32)
    else:
        out_specs = [out_spec, out_spec]
        out_shape = [jax.ShapeDtypeStruct((NP, D), jnp.float32)] * 2
    return pl.pallas_call(
        functools.partial(_lfa_body, has_mlp, is_last),
        grid=(GRID,),
        in_specs=specs,
        out_specs=out_specs,
        out_shape=out_shape,
    )(h, xp, gxp, g10, tbl, *ws)


def kernel(x, xyz, knn, g_pos, ne_W1, ne_g1, ne_b1, ne_W2, ne_g2, ne_b2,
           ne_W3, nbr_g, nbr_b, gpe_W, mlp_Wi, mlp_bi, mlp_Wo, mlp_g, mlp_b,
           lfa_proj, lfa_g, lfa_b, nca_W1, nca_b1, nca_W2, nca_b2, nca_W3a,
           nca_b3a, nca_W3b, nca_b3b, mlps_Wi, mlps_bi, mlps_Wo, mlps_g,
           mlps_b, post_g, post_b, post_W):
    f32 = jnp.float32
    padn = NP - N
    tbl = jnp.concatenate([xyz, x, jnp.zeros((N, 6), f32)], axis=1)
    tbl = jnp.pad(tbl, ((0, padn), (0, 0)))
    knnf = jnp.pad(knn, ((0, padn), (0, 0))).reshape(E // 80, 80).astype(jnp.int32)
    gpos = jnp.pad(g_pos, ((0, padn), (0, 0)))

    r = lambda v: v.reshape(1, -1)

    tblw = jnp.pad(tbl, ((0, 0), (0, D - 16)))
    g10 = _sc_gather(tblw, knnf, D)[:, :16]

    W1s = jnp.pad(ne_W1, ((0, 6), (0, 0)))
    W1d = jnp.pad(ne_W1[:3], ((0, 13), (0, 0)))
    h, xp = _pre_call(
        g10, tbl, gpos, W1s, W1d, r(ne_g1), r(ne_b1), ne_W2, r(ne_g2),
        r(ne_b2), ne_W3, r(nbr_g), r(nbr_b), gpe_W, mlp_Wi, r(mlp_bi),
        mlp_Wo, r(mlp_g), r(mlp_b), lfa_proj[0])

    out = None
    for i in range(4):
        gxp = _sc_gather(xp, knnf, D)
        nW1s = jnp.pad(nca_W1[i], ((0, 13), (0, 0)))
        args = [h, xp, gxp, g10, tbl, nW1s, nW1s, r(nca_b1[i]), nca_W2[i],
                r(nca_b2[i]), nca_W3a[i][:64], nca_W3a[i][64:],
                r(nca_b3a[i]), nca_W3b[i], r(nca_b3b[i]), r(lfa_g[i]),
                r(lfa_b[i])]
        has_mlp = i % 2 == 1
        is_last = i == 3
        if has_mlp:
            j = i // 2
            args += [mlps_Wi[j], r(mlps_bi[j]), mlps_Wo[j], r(mlps_g[j]),
                     r(mlps_b[j])]
        if is_last:
            args += [r(post_g), r(post_b), post_W]
            out = _lfa_call(has_mlp, is_last, *args)
        else:
            args += [lfa_proj[i + 1]]
            h, xp = _lfa_call(has_mlp, is_last, *args)

    return out[:N]


# speedup vs baseline: 2.2392x; 1.0033x over previous
import functools
import math

import jax
import jax.numpy as jnp
from jax import lax
from jax.experimental import pallas as pl
from jax.experimental.pallas import tpu as pltpu
from jax.experimental.pallas import tpu_sc as plsc

N = 10000
K = 16
D = 128
NP = 10240
E = NP * K
B = 256
BK = B * K
GRID = NP // B

_BN_S = 1.0 / math.sqrt(1.0 + 1e-5)
_INV_SQRT2 = 1.0 / math.sqrt(2.0)


def _gelu(x):
    return 0.5 * x * (1.0 + lax.erf(x * _INV_SQRT2))


def _bn(x, g_ref, b_ref):
    return x * (g_ref[...] * _BN_S) + b_ref[...]


_N_CH0 = 96
_N_CH1 = 32


def _sc_gather(table, idx2d, C):
    info = plsc.get_sparse_core_info()
    ns = info.num_subcores
    ch = 80
    nb = 8
    n_max = max(_N_CH0, _N_CH1)
    mesh = plsc.VectorSubcoreMesh(core_axis_name="c", subcore_axis_name="s")

    def body(table_hbm, idx_hbm, out_hbm, idx_v, rows_v, sem_g, sem_o):
        cid = lax.axis_index("c")
        sid = lax.axis_index("s")

        def pipeline(n_ch, chunk0):
            n_grp = n_ch // nb
            pltpu.sync_copy(idx_hbm.at[pl.ds(chunk0, n_ch)],
                            idx_v.at[pl.ds(0, n_ch)])
            base = chunk0 * ch
            for b in range(nb):
                pltpu.async_copy(table_hbm.at[idx_v.at[b]], rows_v.at[b],
                                 sem_g.at[b])

            def grp(g, carry):
                for b in range(nb):
                    c = g * nb + b
                    pltpu.make_async_copy(table_hbm.at[idx_v.at[b]],
                                          rows_v.at[b], sem_g.at[b]).wait()
                    pltpu.async_copy(rows_v.at[b],
                                     out_hbm.at[pl.ds(base + c * ch, ch)],
                                     sem_o.at[b])

                    @pl.when(g < n_grp - 1)
                    def _():
                        pltpu.make_async_copy(
                            rows_v.at[b], out_hbm.at[pl.ds(base, ch)],
                            sem_o.at[b]).wait()
                        pltpu.async_copy(table_hbm.at[idx_v.at[c + nb]],
                                         rows_v.at[b], sem_g.at[b])
                return carry

            lax.fori_loop(0, n_grp, grp, 0)
            for b in range(nb):
                pltpu.make_async_copy(rows_v.at[b],
                                      out_hbm.at[pl.ds(base, ch)],
                                      sem_o.at[b]).wait()

        @pl.when(cid == 0)
        def _():
            pipeline(_N_CH0, sid * _N_CH0)

        @pl.when(cid == 1)
        def _():
            pipeline(_N_CH1, ns * _N_CH0 + sid * _N_CH1)

    fn = pl.kernel(
        body,
        out_type=jax.ShapeDtypeStruct((E, C), jnp.float32),
        mesh=mesh,
        scratch_types=[
            pltpu.VMEM((n_max, ch), jnp.int32),
            pltpu.VMEM((nb, ch, C), jnp.float32),
            pltpu.SemaphoreType.DMA((nb,)),
            pltpu.SemaphoreType.DMA((nb,)),
        ],
    )
    return fn(table, idx2d)


def _rep(x, c):
    return jnp.broadcast_to(x[:, None, :], (B, K, c)).reshape(BK, c)


def _pre_body(g10, tbl, gpos, W1s, W1d, g1, b1, W2, g2, b2, W3, nbrg, nbrb,
              gpeW, mWi, mbi, mWo, mg, mb, proj0, h_out, xp_out):
    e = g10[...]
    h1 = jnp.dot(e, W1s[...], preferred_element_type=jnp.float32)
    dd = jnp.dot(tbl[...], W1d[...], preferred_element_type=jnp.float32)
    h1 = h1 - _rep(dd, 16)
    h1 = _gelu(_bn(h1, g1, b1))
    h2 = jnp.dot(h1, W2[...], preferred_element_type=jnp.float32)
    h2 = _gelu(_bn(h2, g2, b2))
    h3 = jnp.dot(h2, W3[...], preferred_element_type=jnp.float32)
    feat = jnp.max(h3.reshape(B, K, D), axis=1)
    feat = _bn(feat, nbrg, nbrb)
    h = feat + jnp.dot(gpos[...], gpeW[...], preferred_element_type=jnp.float32)
    t = _gelu(jnp.dot(h, mWi[...], preferred_element_type=jnp.float32) + mbi[...])
    h = h + _bn(jnp.dot(t, mWo[...], preferred_element_type=jnp.float32), mg, mb)
    h_out[...] = h
    xp_out[...] = jnp.dot(h, proj0[...], preferred_element_type=jnp.float32)


def _full(a):
    return pl.BlockSpec(a.shape, lambda i: (0,) * a.ndim)


def _pre_call(g10, tbl, gpos, *ws):
    specs = [
        pl.BlockSpec((BK, 16), lambda i: (i, 0)),
        pl.BlockSpec((B, 16), lambda i: (i, 0)),
        pl.BlockSpec((B, 64), lambda i: (i, 0)),
    ] + [_full(w) for w in ws]
    out_spec = pl.BlockSpec((B, D), lambda i: (i, 0))
    return pl.pallas_call(
        _pre_body,
        grid=(GRID,),
        in_specs=specs,
        out_specs=[out_spec, out_spec],
        out_shape=[jax.ShapeDtypeStruct((NP, D), jnp.float32)] * 2,
    )(g10, tbl, gpos, *ws)


def _lfa_body(has_mlp, is_last, h_in, xp_in, gxp, g10, tbl,
              W1s, W1d, b1, W2i, b2, W3at, W3ab, b3a, W3b, b3b, lg, lb,
              *rest):
    if has_mlp:
        msWi, msbi, msWo, msg, msb = rest[:5]
        rest = rest[5:]
    if is_last:
        postg, postb, postW = rest[:3]
        out_ref = rest[3]
    else:
        projn = rest[0]
        h_out, xp_out = rest[1], rest[2]

    e = g10[...]
    p0 = jnp.dot(e, W1s[...], preferred_element_type=jnp.float32) + b1[...]
    dd = jnp.dot(tbl[...], W1d[...], preferred_element_type=jnp.float32)
    p0 = p0 - _rep(dd, 64)
    plo = jnp.max(p0.reshape(B, K, 64), axis=1)
    p1 = jnp.dot(p0, W2i[...], preferred_element_type=jnp.float32) + b2[...]
    q = jnp.dot(plo, W3ab[...], preferred_element_type=jnp.float32)
    a = jnp.dot(p1, W3at[...], preferred_element_type=jnp.float32)
    a = a + _rep(q, D) + b3a[...]
    pe = jnp.dot(_gelu(a), W3b[...], preferred_element_type=jnp.float32) + b3b[...]
    m = jnp.max((gxp[...] + pe).reshape(B, K, D), axis=1)
    h = h_in[...] + _bn(m - xp_in[...], lg, lb)
    if has_mlp:
        t = _gelu(jnp.dot(h, msWi[...], preferred_element_type=jnp.float32) + msbi[...])
        h = h + _bn(jnp.dot(t, msWo[...], preferred_element_type=jnp.float32), msg, msb)
    if is_last:
        out_ref[...] = jnp.dot(_bn(h, postg, postb), postW[...],
                               preferred_element_type=jnp.float32)
    else:
        h_out[...] = h
        xp_out[...] = jnp.dot(h, projn[...], preferred_element_type=jnp.float32)


def _lfa_call(has_mlp, is_last, h, xp, gxp, g10, tbl, *ws):
    specs = [
        pl.BlockSpec((B, D), lambda i: (i, 0)),
        pl.BlockSpec((B, D), lambda i: (i, 0)),
        pl.BlockSpec((BK, D), lambda i: (i, 0)),
        pl.BlockSpec((BK, 16), lambda i: (i, 0)),
        pl.BlockSpec((B, 16), lambda i: (i, 0)),
    ] + [_full(w) for w in ws]
    out_spec = pl.BlockSpec((B, D), lambda i: (i, 0))
    if is_last:
        out_specs = out_spec
        out_shape = jax.ShapeDtypeStruct((NP, D), jnp.float32)
    else:
        out_specs = [out_spec, out_spec]
        out_shape = [jax.ShapeDtypeStruct((NP, D), jnp.float32)] * 2
    return pl.pallas_call(
        functools.partial(_lfa_body, has_mlp, is_last),
        grid=(GRID,),
        in_specs=specs,
        out_specs=out_specs,
        out_shape=out_shape,
    )(h, xp, gxp, g10, tbl, *ws)


def kernel(x, xyz, knn, g_pos, ne_W1, ne_g1, ne_b1, ne_W2, ne_g2, ne_b2,
           ne_W3, nbr_g, nbr_b, gpe_W, mlp_Wi, mlp_bi, mlp_Wo, mlp_g, mlp_b,
           lfa_proj, lfa_g, lfa_b, nca_W1, nca_b1, nca_W2, nca_b2, nca_W3a,
           nca_b3a, nca_W3b, nca_b3b, mlps_Wi, mlps_bi, mlps_Wo, mlps_g,
           mlps_b, post_g, post_b, post_W):
    f32 = jnp.float32
    padn = NP - N
    tbl = jnp.concatenate([xyz, x, jnp.zeros((N, 6), f32)], axis=1)
    tbl = jnp.pad(tbl, ((0, padn), (0, 0)))
    knnf = jnp.pad(knn, ((0, padn), (0, 0))).reshape(E // 80, 80).astype(jnp.int32)
    gpos = jnp.pad(g_pos, ((0, padn), (0, 0)))

    r = lambda v: v.reshape(1, -1)

    tblw = jnp.pad(tbl, ((0, 0), (0, D - 16)))
    g10 = _sc_gather(tblw, knnf, D)[:, :16]

    W1s = jnp.pad(ne_W1, ((0, 6), (0, 0)))
    W1d = jnp.pad(ne_W1[:3], ((0, 13), (0, 0)))
    h, xp = _pre_call(
        g10, tbl, gpos, W1s, W1d, r(ne_g1), r(ne_b1), ne_W2, r(ne_g2),
        r(ne_b2), ne_W3, r(nbr_g), r(nbr_b), gpe_W, mlp_Wi, r(mlp_bi),
        mlp_Wo, r(mlp_g), r(mlp_b), lfa_proj[0])

    out = None
    for i in range(4):
        gxp = _sc_gather(xp, knnf, D)
        nW1s = jnp.pad(nca_W1[i], ((0, 13), (0, 0)))
        args = [h, xp, gxp, g10, tbl, nW1s, nW1s, r(nca_b1[i]), nca_W2[i],
                r(nca_b2[i]), nca_W3a[i][:64], nca_W3a[i][64:],
                r(nca_b3a[i]), nca_W3b[i], r(nca_b3b[i]), r(lfa_g[i]),
                r(lfa_b[i])]
        has_mlp = i % 2 == 1
        is_last = i == 3
        if has_mlp:
            j = i // 2
            args += [mlps_Wi[j], r(mlps_bi[j]), mlps_Wo[j], r(mlps_g[j]),
                     r(mlps_b[j])]
        if is_last:
            args += [r(post_g), r(post_b), post_W]
            out = _lfa_call(has_mlp, is_last, *args)
        else:
            args += [lfa_proj[i + 1]]
            h, xp = _lfa_call(has_mlp, is_last, *args)

    return out[:N]
